# Initial kernel scaffold; baseline (speedup 1.0000x reference)
import jax
import jax.numpy as jnp
from jax.experimental import pallas as pl


def kernel(nodes, adj, x, W):
    raise NotImplementedError("write your pallas kernel here")


# speedup vs baseline: 1.2088x; 1.2088x over previous
import jax
import jax.numpy as jnp
import numpy as np
from jax import lax
from jax.experimental import pallas as pl
from jax.experimental.pallas import tpu as pltpu
from jax.experimental.pallas import tpu_sc as plsc

N_NODES = 100000
D = 128
B = 10000
K = 10
F = K + 1

NC, NS, L = 2, 16, 16
NW = NC * NS
B_PAD = 10240
BPW = B_PAD // NW
C = 8
STEPS = BPW // C
ROWS_PER_STEP = C * F
NVREG = D // L
NIDX = BPW * F

_INV_DENOM = 1.0 / 11.0


def _sc_body(all_idx_hbm, x_hbm, agg_hbm,
             all_idx_v, rows_v, out_v, sem_x):
    wid = lax.axis_index("s") * NC + lax.axis_index("c")
    pltpu.sync_copy(all_idx_hbm.at[pl.ds(wid * NIDX, NIDX)], all_idx_v)

    def step(s, carry):
        d = pltpu.async_copy(
            x_hbm.at[all_idx_v.at[pl.ds(s * ROWS_PER_STEP, ROWS_PER_STEP)]],
            rows_v, sem_x)
        d.wait()
        for i in range(C):
            for v in range(NVREG):
                acc = rows_v[i * F, pl.ds(v * L, L)]
                for j in range(1, F):
                    acc = acc + rows_v[i * F + j, pl.ds(v * L, L)]
                out_v[s * C + i, pl.ds(v * L, L)] = acc * _INV_DENOM
        return carry

    lax.fori_loop(0, STEPS, step, 0)

    pltpu.sync_copy(out_v, agg_hbm.at[pl.ds(wid * BPW, BPW)])


@jax.jit
def _sc_aggregate(all_idx, x):
    mesh = plsc.VectorSubcoreMesh(core_axis_name="c", subcore_axis_name="s")
    return pl.kernel(
        _sc_body,
        out_type=jax.ShapeDtypeStruct((B_PAD, D), jnp.float32),
        mesh=mesh,
        scratch_types=[
            pltpu.VMEM((NIDX,), jnp.int32),
            pltpu.VMEM((ROWS_PER_STEP, D), jnp.float32),
            pltpu.VMEM((BPW, D), jnp.float32),
            pltpu.SemaphoreType.DMA,
        ],
    )(all_idx, x)


def _mm_body(a_ref, wt_ref, o_ref):
    o_ref[...] = jnp.maximum(
        jnp.dot(a_ref[...], wt_ref[...], preferred_element_type=jnp.float32),
        0.0)


MM_BLOCK = 400


@jax.jit
def _tc_matmul_relu(agg_pad, Wt):
    return pl.pallas_call(
        _mm_body,
        grid=(B // MM_BLOCK,),
        in_specs=[
            pl.BlockSpec((MM_BLOCK, D), lambda i: (i, 0)),
            pl.BlockSpec((D, D), lambda i: (0, 0)),
        ],
        out_specs=pl.BlockSpec((MM_BLOCK, D), lambda i: (i, 0)),
        out_shape=jax.ShapeDtypeStruct((B, D), jnp.float32),
    )(agg_pad, Wt)


def kernel(nodes, adj, x, W):
    nodes_pad = jnp.pad(nodes, (0, B_PAD - B))
    all_idx = jnp.concatenate(
        [nodes_pad[:, None], jnp.take(adj, nodes_pad, axis=0)],
        axis=1).reshape(-1)
    agg_pad = _sc_aggregate(all_idx, x)
    return _tc_matmul_relu(agg_pad, W.T)


# speedup vs baseline: 1.4178x; 1.1729x over previous
import jax
import jax.numpy as jnp
from jax import lax
from jax.experimental import pallas as pl
from jax.experimental.pallas import tpu as pltpu
from jax.experimental.pallas import tpu_sc as plsc

N_NODES = 100000
D = 128
B = 10000
K = 10
F = K + 1

NC, NS, L = 2, 16, 16
NW = NC * NS
B_PAD = 10240
BPW = B_PAD // NW
C = 8
STEPS = BPW // C
RPS = C * F
NVREG = D // L
NIDX = BPW * F

_INV_DENOM = 1.0 / 11.0


def _sc_body(all_idx_hbm, x_hbm, agg_hbm,
             all_idx_v, rows0, rows1, out_v, sem0, sem1, sem_w):
    wid = lax.axis_index("s") * NC + lax.axis_index("c")
    pltpu.sync_copy(all_idx_hbm.at[pl.ds(wid * NIDX, NIDX)], all_idx_v)

    def issue(s, buf, sem):
        pltpu.async_copy(
            x_hbm.at[all_idx_v.at[pl.ds(s * RPS, RPS)]], buf, sem)

    def drain(buf, sem):
        pltpu.make_async_copy(x_hbm.at[pl.ds(0, RPS)], buf, sem).wait()

    def compute(s, buf):
        for i in range(C):
            for v in range(NVREG):
                acc = buf[i * F, pl.ds(v * L, L)]
                for j in range(1, F):
                    acc = acc + buf[i * F + j, pl.ds(v * L, L)]
                out_v[s * C + i, pl.ds(v * L, L)] = acc * _INV_DENOM
        pltpu.async_copy(
            out_v.at[pl.ds(s * C, C)],
            agg_hbm.at[pl.ds(wid * BPW + s * C, C)], sem_w)

    issue(0, rows0, sem0)

    def body(t, carry):
        issue(2 * t + 1, rows1, sem1)
        drain(rows0, sem0)
        compute(2 * t, rows0)

        @pl.when(t < STEPS // 2 - 1)
        def _():
            issue(2 * t + 2, rows0, sem0)

        drain(rows1, sem1)
        compute(2 * t + 1, rows1)
        return carry

    lax.fori_loop(0, STEPS // 2, body, 0)

    pltpu.make_async_copy(
        out_v, agg_hbm.at[pl.ds(wid * BPW, BPW)], sem_w).wait()


@jax.jit
def _sc_aggregate(all_idx, x):
    mesh = plsc.VectorSubcoreMesh(core_axis_name="c", subcore_axis_name="s")
    return pl.kernel(
        _sc_body,
        out_type=jax.ShapeDtypeStruct((B_PAD, D), jnp.float32),
        mesh=mesh,
        scratch_types=[
            pltpu.VMEM((NIDX,), jnp.int32),
            pltpu.VMEM((RPS, D), jnp.float32),
            pltpu.VMEM((RPS, D), jnp.float32),
            pltpu.VMEM((BPW, D), jnp.float32),
            pltpu.SemaphoreType.DMA,
            pltpu.SemaphoreType.DMA,
            pltpu.SemaphoreType.DMA,
        ],
    )(all_idx, x)


def _mm_body(a_ref, wt_ref, o_ref):
    o_ref[...] = jnp.maximum(
        jnp.dot(a_ref[...], wt_ref[...], preferred_element_type=jnp.float32),
        0.0)


MM_BLOCK = 400


@jax.jit
def _tc_matmul_relu(agg_pad, Wt):
    return pl.pallas_call(
        _mm_body,
        grid=(B // MM_BLOCK,),
        in_specs=[
            pl.BlockSpec((MM_BLOCK, D), lambda i: (i, 0)),
            pl.BlockSpec((D, D), lambda i: (0, 0)),
        ],
        out_specs=pl.BlockSpec((MM_BLOCK, D), lambda i: (i, 0)),
        out_shape=jax.ShapeDtypeStruct((B, D), jnp.float32),
    )(agg_pad, Wt)


def kernel(nodes, adj, x, W):
    nodes_pad = jnp.pad(nodes, (0, B_PAD - B))
    all_idx = jnp.concatenate(
        [nodes_pad[:, None], jnp.take(adj, nodes_pad, axis=0)],
        axis=1).reshape(-1)
    agg_pad = _sc_aggregate(all_idx, x)
    return _tc_matmul_relu(agg_pad, W.T)


# speedup vs baseline: 1.8737x; 1.3216x over previous
import jax
import jax.numpy as jnp
from jax import lax
from jax.experimental import pallas as pl
from jax.experimental.pallas import tpu as pltpu
from jax.experimental.pallas import tpu_sc as plsc

N_NODES = 100000
D = 128
B = 10000
K = 10
F = K + 1

NC, NS, L = 2, 16, 16
NW = NC * NS
B_PAD = 10240
BPW = B_PAD // NW
C = 8
STEPS = BPW // C
RPS = C * F
NVREG = D // L
NIDX = BPW * F
NBUF = 4

_INV_DENOM = 1.0 / 11.0


def _sc_body(all_idx_hbm, x_hbm, agg_hbm,
             all_idx_v, rows0, rows1, rows2, rows3,
             sem0, sem1, sem2, sem3, out_v, sem_w):
    bufs = (rows0, rows1, rows2, rows3)
    sems = (sem0, sem1, sem2, sem3)
    wid = lax.axis_index("s") * NC + lax.axis_index("c")
    pltpu.sync_copy(all_idx_hbm.at[pl.ds(wid * NIDX, NIDX)], all_idx_v)

    def issue(s, b):
        pltpu.async_copy(
            x_hbm.at[all_idx_v.at[pl.ds(s * RPS, RPS)]], bufs[b], sems[b])

    def drain(b):
        pltpu.make_async_copy(x_hbm.at[pl.ds(0, RPS)], bufs[b], sems[b]).wait()

    def compute(s, b):
        buf = bufs[b]

        def node(i, carry):
            for v in range(NVREG):
                acc = buf[i * F, pl.ds(v * L, L)]
                for j in range(1, F):
                    acc = acc + buf[i * F + j, pl.ds(v * L, L)]
                out_v[s * C + i, pl.ds(v * L, L)] = acc * _INV_DENOM
            return carry

        lax.fori_loop(0, C, node, 0)
        pltpu.async_copy(
            out_v.at[pl.ds(s * C, C)],
            agg_hbm.at[pl.ds(wid * BPW + s * C, C)], sem_w)

    for b in range(NBUF - 1):
        issue(b, b)

    def body(t, carry):
        s0 = NBUF * t
        issue(s0 + NBUF - 1, NBUF - 1)
        for b in range(NBUF):
            drain(b)
            compute(s0 + b, b)
            if b < NBUF - 1:
                @pl.when(s0 + NBUF + b < STEPS)
                def _():
                    issue(s0 + NBUF + b, b)
        return carry

    lax.fori_loop(0, STEPS // NBUF, body, 0)

    pltpu.make_async_copy(
        out_v, agg_hbm.at[pl.ds(wid * BPW, BPW)], sem_w).wait()


@jax.jit
def _sc_aggregate(all_idx, x):
    mesh = plsc.VectorSubcoreMesh(core_axis_name="c", subcore_axis_name="s")
    return pl.kernel(
        _sc_body,
        out_type=jax.ShapeDtypeStruct((B_PAD, D), jnp.float32),
        mesh=mesh,
        scratch_types=[
            pltpu.VMEM((NIDX,), jnp.int32),
            pltpu.VMEM((RPS, D), jnp.float32),
            pltpu.VMEM((RPS, D), jnp.float32),
            pltpu.VMEM((RPS, D), jnp.float32),
            pltpu.VMEM((RPS, D), jnp.float32),
            pltpu.SemaphoreType.DMA,
            pltpu.SemaphoreType.DMA,
            pltpu.SemaphoreType.DMA,
            pltpu.SemaphoreType.DMA,
            pltpu.VMEM((BPW, D), jnp.float32),
            pltpu.SemaphoreType.DMA,
        ],
    )(all_idx, x)


def _mm_body(a_ref, wt_ref, o_ref):
    o_ref[...] = jnp.maximum(
        jnp.dot(a_ref[...], wt_ref[...], preferred_element_type=jnp.float32),
        0.0)


MM_BLOCK = 400


@jax.jit
def _tc_matmul_relu(agg_pad, Wt):
    return pl.pallas_call(
        _mm_body,
        grid=(B // MM_BLOCK,),
        in_specs=[
            pl.BlockSpec((MM_BLOCK, D), lambda i: (i, 0)),
            pl.BlockSpec((D, D), lambda i: (0, 0)),
        ],
        out_specs=pl.BlockSpec((MM_BLOCK, D), lambda i: (i, 0)),
        out_shape=jax.ShapeDtypeStruct((B, D), jnp.float32),
    )(agg_pad, Wt)


def kernel(nodes, adj, x, W):
    nodes_pad = jnp.pad(nodes, (0, B_PAD - B))
    all_idx = jnp.concatenate(
        [nodes_pad[:, None], jnp.take(adj, nodes_pad, axis=0)],
        axis=1).reshape(-1)
    agg_pad = _sc_aggregate(all_idx, x)
    return _tc_matmul_relu(agg_pad, W.T)
